# baseline (speedup 1.0000x reference)
import jax
import jax.numpy as jnp
from jax.experimental import pallas as pl

B, N, D = 8, 512, 64
OUT = 64
NUM_LAYERS = 2
GPB = 2
GRID = B // GPB


def _split(a):
    hi = a.astype(jnp.bfloat16)
    lo = (a - hi.astype(jnp.float32)).astype(jnp.bfloat16)
    return hi, lo


def _dot(a, b, dn):
    return jax.lax.dot_general(a, b, (dn, ((), ())),
                               preferred_element_type=jnp.float32)


def _ggnn_kernel(x_ref, adj_ref, w_ref, w_ih_ref, w_hh_ref, b_ih_ref,
                 b_hh_ref, out_ref):
    b_ih = b_ih_ref[0][None, :]
    b_hh = b_hh_ref[0][None, :]
    w_ih_bf = w_ih_ref[...].astype(jnp.bfloat16)
    w_hh_bf = w_hh_ref[...].astype(jnp.bfloat16)
    w_bf = [w_ref[l].astype(jnp.bfloat16) for l in range(NUM_LAYERS)]

    H = x_ref[...].reshape(GPB * N, D)
    adj_bf = adj_ref[...].astype(jnp.bfloat16)

    for layer in range(NUM_LAYERS):
        H_bf = H.astype(jnp.bfloat16)
        T = jax.lax.dot_general(
            adj_bf, H_bf.reshape(GPB, N, D),
            (((1,), (1,)), ((0,), (0,))),
            preferred_element_type=jnp.float32)
        TH, TL = _split(T.reshape(GPB * N, D))
        S = jnp.concatenate([TH, TL], axis=0)
        A = _dot(S, w_bf[layer], ((1,), (0,)))
        agg = A[:GPB * N] + A[GPB * N:]
        gi = _dot(agg.astype(jnp.bfloat16), w_ih_bf, ((1,), (1,))) + b_ih
        gh = _dot(H_bf, w_hh_bf, ((1,), (1,))) + b_hh
        i_r, i_z, i_n = gi[:, :OUT], gi[:, OUT:2 * OUT], gi[:, 2 * OUT:]
        h_r, h_z, h_n = gh[:, :OUT], gh[:, OUT:2 * OUT], gh[:, 2 * OUT:]
        r = jax.nn.sigmoid(i_r + h_r)
        z = jax.nn.sigmoid(i_z + h_z)
        n = jnp.tanh(i_n + r * h_n)
        H = n + z * (H - n)

    out_ref[...] = H.reshape(GPB, N, OUT)


def kernel(x, adj, W, w_ih, w_hh, b_ih, b_hh):
    out = pl.pallas_call(
        _ggnn_kernel,
        grid=(GRID,),
        in_specs=[
            pl.BlockSpec((GPB, N, D), lambda b: (b, 0, 0)),
            pl.BlockSpec((GPB, N, N), lambda b: (b, 0, 0)),
            pl.BlockSpec((NUM_LAYERS, OUT, OUT), lambda b: (0, 0, 0)),
            pl.BlockSpec((3 * OUT, OUT), lambda b: (0, 0)),
            pl.BlockSpec((3 * OUT, OUT), lambda b: (0, 0)),
            pl.BlockSpec((1, 3 * OUT), lambda b: (0, 0)),
            pl.BlockSpec((1, 3 * OUT), lambda b: (0, 0)),
        ],
        out_specs=pl.BlockSpec((GPB, N, OUT), lambda b: (b, 0, 0)),
        out_shape=jax.ShapeDtypeStruct((B, N, OUT), jnp.float32),
    )(x, adj, W, w_ih, w_hh, b_ih.reshape(1, -1), b_hh.reshape(1, -1))
    return out


# speedup vs baseline: 1.0756x; 1.0756x over previous
import jax
import jax.numpy as jnp
from jax.experimental import pallas as pl

B, N, D = 8, 512, 64
OUT = 64
NUM_LAYERS = 2
GPB = 4
GRID = B // GPB


def _split(a):
    hi = a.astype(jnp.bfloat16)
    lo = (a - hi.astype(jnp.float32)).astype(jnp.bfloat16)
    return hi, lo


def _dot(a, b, dn):
    return jax.lax.dot_general(a, b, (dn, ((), ())),
                               preferred_element_type=jnp.float32)


def _ggnn_kernel(x_ref, adj_ref, w_ref, w_ih_ref, w_hh_ref, b_ih_ref,
                 b_hh_ref, out_ref):
    b_ih = b_ih_ref[0][None, :]
    b_hh = b_hh_ref[0][None, :]
    w_ih_bf = w_ih_ref[...].astype(jnp.bfloat16)
    w_hh_bf = w_hh_ref[...].astype(jnp.bfloat16)
    w_bf = [w_ref[l].astype(jnp.bfloat16) for l in range(NUM_LAYERS)]

    H = x_ref[...].reshape(GPB * N, D)
    adj_bf = adj_ref[...].astype(jnp.bfloat16)

    for layer in range(NUM_LAYERS):
        H_bf = H.astype(jnp.bfloat16)
        T = jax.lax.dot_general(
            adj_bf, H_bf.reshape(GPB, N, D),
            (((1,), (1,)), ((0,), (0,))),
            preferred_element_type=jnp.float32)
        TH, TL = _split(T.reshape(GPB * N, D))
        S = jnp.concatenate([TH, TL], axis=0)
        A = _dot(S, w_bf[layer], ((1,), (0,)))
        agg = A[:GPB * N] + A[GPB * N:]
        gi = _dot(agg.astype(jnp.bfloat16), w_ih_bf, ((1,), (1,))) + b_ih
        gh = _dot(H_bf, w_hh_bf, ((1,), (1,))) + b_hh
        i_r, i_z, i_n = gi[:, :OUT], gi[:, OUT:2 * OUT], gi[:, 2 * OUT:]
        h_r, h_z, h_n = gh[:, :OUT], gh[:, OUT:2 * OUT], gh[:, 2 * OUT:]
        r = jax.nn.sigmoid(i_r + h_r)
        z = jax.nn.sigmoid(i_z + h_z)
        n = jnp.tanh(i_n + r * h_n)
        H = n + z * (H - n)

    out_ref[...] = H.reshape(GPB, N, OUT)


def kernel(x, adj, W, w_ih, w_hh, b_ih, b_hh):
    out = pl.pallas_call(
        _ggnn_kernel,
        grid=(GRID,),
        in_specs=[
            pl.BlockSpec((GPB, N, D), lambda b: (b, 0, 0)),
            pl.BlockSpec((GPB, N, N), lambda b: (b, 0, 0)),
            pl.BlockSpec((NUM_LAYERS, OUT, OUT), lambda b: (0, 0, 0)),
            pl.BlockSpec((3 * OUT, OUT), lambda b: (0, 0)),
            pl.BlockSpec((3 * OUT, OUT), lambda b: (0, 0)),
            pl.BlockSpec((1, 3 * OUT), lambda b: (0, 0)),
            pl.BlockSpec((1, 3 * OUT), lambda b: (0, 0)),
        ],
        out_specs=pl.BlockSpec((GPB, N, OUT), lambda b: (b, 0, 0)),
        out_shape=jax.ShapeDtypeStruct((B, N, OUT), jnp.float32),
    )(x, adj, W, w_ih, w_hh, b_ih.reshape(1, -1), b_hh.reshape(1, -1))
    return out
